# Initial kernel scaffold; baseline (speedup 1.0000x reference)
import jax
import jax.numpy as jnp
from jax.experimental import pallas as pl


def kernel(batch_reprs, batch_labels):
    raise NotImplementedError("write your pallas kernel here")


# speedup vs baseline: 1.8909x; 1.8909x over previous
import jax
import jax.numpy as jnp
from jax.experimental import pallas as pl
from jax.experimental.pallas import tpu as pltpu

L = 4096
D = 64
BLOCK_R = 256
N_SUBSTAGES = 78
MARGIN = 0.2


def _sortable_key(v, match_i32):
    b = jax.lax.bitcast_convert_type(v, jnp.int32)
    key = b ^ ((b >> 31) & jnp.int32(0x7FFFFFFF))
    return (key & jnp.int32(~1)) | match_i32


def _decode_key(key2):
    m = key2 & jnp.int32(1)
    kr = key2 & jnp.int32(~1)
    vb = kr ^ ((kr >> 31) & jnp.int32(0x7FFFFFFF))
    return jax.lax.bitcast_convert_type(vb, jnp.float32), m


def _bitonic_desc(a, iota):
    n = a.shape[1]

    def body(_, carry):
        a, j, k = carry
        s0 = (iota & j) == 0
        left = pltpu.roll(a, n - j, axis=1)
        right = pltpu.roll(a, j, axis=1)
        z = jnp.where(s0, left, right)
        desc = (iota & k) == 0
        want_max = s0 == desc
        a = jnp.where(want_max, jnp.maximum(a, z), jnp.minimum(a, z))
        j2 = j // 2
        stage_done = j2 == 0
        k2 = jnp.where(stage_done, k * 2, k)
        j2 = jnp.where(stage_done, k2 // 2, j2)
        return a, j2, k2

    a, _, _ = jax.lax.fori_loop(
        0, N_SUBSTAGES, body,
        (a, jnp.int32(1), jnp.int32(2)))
    return a


def _loss_kernel(xb_ref, xa_ref, lb_ref, la_ref, out_ref):
    xb = xb_ref[...]
    xa = xa_ref[...]
    lb = lb_ref[...]
    la = la_ref[...]

    dn = (((1,), (1,)), ((), ()))
    g = jax.lax.dot_general(xb, xa, dn,
                            preferred_element_type=jnp.float32,
                            precision=jax.lax.Precision.HIGHEST)
    ones = jnp.ones((1, D), dtype=jnp.float32)
    sqa = jax.lax.dot_general(ones, xa * xa, dn,
                              preferred_element_type=jnp.float32,
                              precision=jax.lax.Precision.HIGHEST)
    sqb = jnp.sum(xb * xb, axis=1, keepdims=True)

    d2 = jnp.maximum(sqb + sqa - 2.0 * g, 0.0)
    dist = jnp.sqrt(jnp.maximum(d2, 1e-12))
    match = (lb == la)
    vhat = -dist + jnp.where(match, 0.0, MARGIN)

    key2 = _sortable_key(vhat, match.astype(jnp.int32))
    iota = jax.lax.broadcasted_iota(jnp.int32, key2.shape, 1)
    skey = _bitonic_desc(key2, iota)

    vs, mi = _decode_key(skey)
    m = mi.astype(jnp.float32)
    t = (iota + 1).astype(jnp.float32)
    kpos = jnp.sum(m, axis=1, keepdims=True)

    fp_mask = (t <= kpos) & (mi == 0)
    fn_mask = (t > kpos) & (mi == 1)
    fp_w = 0.5 + (kpos - t + 1.0) / kpos * 0.5
    fn_w = 0.5 + (t - kpos) / jnp.maximum(float(L) - kpos, 1.0) * 0.5
    part = (jnp.sum(jnp.where(fp_mask, vs * fp_w, 0.0), keepdims=True)
            - jnp.sum(jnp.where(fn_mask, vs * fn_w, 0.0), keepdims=True))

    @pl.when(pl.program_id(0) == 0)
    def _():
        out_ref[...] = jnp.zeros_like(part)
    out_ref[...] += part


def kernel(batch_reprs, batch_labels):
    x = batch_reprs.astype(jnp.float32)
    lab = batch_labels.astype(jnp.int32)
    lab_col = lab.reshape(L, 1)
    lab_row = lab.reshape(1, L)
    grid = L // BLOCK_R
    out = pl.pallas_call(
        _loss_kernel,
        grid=(grid,),
        in_specs=[
            pl.BlockSpec((BLOCK_R, D), lambda i: (i, 0)),
            pl.BlockSpec((L, D), lambda i: (0, 0)),
            pl.BlockSpec((BLOCK_R, 1), lambda i: (i, 0)),
            pl.BlockSpec((1, L), lambda i: (0, 0)),
        ],
        out_specs=pl.BlockSpec((1, 1), lambda i: (0, 0)),
        out_shape=jax.ShapeDtypeStruct((1, 1), jnp.float32),
    )(x, x, lab_col, lab_row)
    return out[0, 0]
